# baseline (speedup 1.0000x reference)
import functools

import jax
import jax.numpy as jnp
from jax import lax
from jax.experimental import pallas as pl
from jax.experimental.pallas import tpu as pltpu
from jax.experimental.pallas import tpu_sc as plsc

B = 16384
D = 64
NW = 32
BPW = B // NW
CH = 128
NCH = BPW // CH
BT = 2048


def _gather1_body(idx_hbm, table, out, idx_v, buf0, buf1, sem0, sem1):
    wid = lax.axis_index("s") * 2 + lax.axis_index("c")
    pltpu.sync_copy(idx_hbm.at[wid], idx_v)
    base = wid * BPW

    bufs = (buf0, buf1)
    sems = (sem0, sem1)
    prev = pltpu.async_copy(table.at[idx_v.at[0]], bufs[0], sems[0])
    for j in range(1, NCH):
        cur = pltpu.async_copy(table.at[idx_v.at[j]], bufs[j % 2], sems[j % 2])
        prev.wait()
        pltpu.sync_copy(bufs[(j - 1) % 2], out.at[pl.ds(base + (j - 1) * CH, CH)])
        prev = cur
    prev.wait()
    pltpu.sync_copy(bufs[(NCH - 1) % 2], out.at[pl.ds(base + (NCH - 1) * CH, CH)])


@functools.lru_cache(maxsize=2)
def _get_gather1(dtype):
    mesh = plsc.VectorSubcoreMesh(core_axis_name="c", subcore_axis_name="s")
    return pl.kernel(
        _gather1_body,
        mesh=mesh,
        out_type=jax.ShapeDtypeStruct((B, D), dtype),
        scratch_types=[
            pltpu.VMEM((NCH, CH), jnp.int32),
            pltpu.VMEM((CH, D), dtype),
            pltpu.VMEM((CH, D), dtype),
            pltpu.SemaphoreType.DMA,
            pltpu.SemaphoreType.DMA,
        ],
        compiler_params=pltpu.CompilerParams(use_tc_tiling_on_sc=False),
    )


def _fuse_body(ug, ig, um, im, w1a, w1b, b1, wg, wm, bo, out):
    umf = um[...].astype(jnp.float32)
    imf = im[...].astype(jnp.float32)
    h = jnp.dot(umf, w1a[...], preferred_element_type=jnp.float32)
    h = h + jnp.dot(imf, w1b[...], preferred_element_type=jnp.float32)
    h = jnp.maximum(h + b1[...], 0.0)
    g = ug[...].astype(jnp.float32) * ig[...].astype(jnp.float32)
    out[...] = (jnp.dot(g, wg[...], preferred_element_type=jnp.float32)
                + jnp.dot(h, wm[...], preferred_element_type=jnp.float32)
                + bo[...])


@functools.lru_cache(maxsize=1)
def _get_fuse():
    return pl.pallas_call(
        _fuse_body,
        grid=(B // BT,),
        in_specs=[
            pl.BlockSpec((BT, D), lambda i: (i, 0)),
            pl.BlockSpec((BT, D), lambda i: (i, 0)),
            pl.BlockSpec((BT, D), lambda i: (i, 0)),
            pl.BlockSpec((BT, D), lambda i: (i, 0)),
            pl.BlockSpec((D, D), lambda i: (0, 0)),
            pl.BlockSpec((D, D), lambda i: (0, 0)),
            pl.BlockSpec((1, D), lambda i: (0, 0)),
            pl.BlockSpec((D, 1), lambda i: (0, 0)),
            pl.BlockSpec((D, 1), lambda i: (0, 0)),
            pl.BlockSpec((1, 1), lambda i: (0, 0)),
        ],
        out_specs=pl.BlockSpec((BT, 1), lambda i: (i, 0)),
        out_shape=jax.ShapeDtypeStruct((B, 1), jnp.float32),
    )


def kernel(user_indices, item_indices, user_emb_ncf, item_emb_ncf,
           user_emb_mlp, item_emb_mlp, W1, b1, W_out, b_out):
    uidx = user_indices.astype(jnp.int32).reshape(NW, NCH, CH)
    iidx = item_indices.astype(jnp.int32).reshape(NW, NCH, CH)
    g16 = _get_gather1(jnp.bfloat16)
    g32 = _get_gather1(jnp.float32)
    ug = g16(uidx, user_emb_ncf.astype(jnp.bfloat16))
    um = g16(uidx, user_emb_mlp.astype(jnp.bfloat16))
    ig = g32(iidx, item_emb_ncf)
    im = g32(iidx, item_emb_mlp)
    return _get_fuse()(ug, ig, um, im, W1[:D], W1[D:], b1.reshape(1, D),
                       W_out[:D], W_out[D:], b_out.reshape(1, 1))


# speedup vs baseline: 1.5307x; 1.5307x over previous
import functools

import jax
import jax.numpy as jnp
from jax import lax
from jax.experimental import pallas as pl
from jax.experimental.pallas import tpu as pltpu
from jax.experimental.pallas import tpu_sc as plsc

B = 16384
D = 64
D2 = 128
NW = 32
BPW = B // NW
CH = 128
NCH = BPW // CH
BT = 2048
CT = 1024


def _packT_body(ta, tb, out):
    out[...] = jnp.concatenate((ta[...].T, tb[...].T), axis=1)


@functools.lru_cache(maxsize=4)
def _get_packT(n):
    return pl.pallas_call(
        _packT_body,
        grid=(pl.cdiv(n, CT),),
        in_specs=[
            pl.BlockSpec((D, CT), lambda i: (0, i)),
            pl.BlockSpec((D, CT), lambda i: (0, i)),
        ],
        out_specs=pl.BlockSpec((CT, D2), lambda i: (i, 0)),
        out_shape=jax.ShapeDtypeStruct((n, D2), jnp.float32),
    )


def _gather1_body(idx_hbm, table, out, idx_v, buf0, buf1, sem0, sem1):
    wid = lax.axis_index("s") * 2 + lax.axis_index("c")
    pltpu.sync_copy(idx_hbm.at[wid], idx_v)
    base = wid * BPW

    bufs = (buf0, buf1)
    sems = (sem0, sem1)
    prev = pltpu.async_copy(table.at[idx_v.at[0]], bufs[0], sems[0])
    for j in range(1, NCH):
        cur = pltpu.async_copy(table.at[idx_v.at[j]], bufs[j % 2], sems[j % 2])
        prev.wait()
        pltpu.sync_copy(bufs[(j - 1) % 2], out.at[pl.ds(base + (j - 1) * CH, CH)])
        prev = cur
    prev.wait()
    pltpu.sync_copy(bufs[(NCH - 1) % 2], out.at[pl.ds(base + (NCH - 1) * CH, CH)])


@functools.lru_cache(maxsize=1)
def _get_gather1():
    mesh = plsc.VectorSubcoreMesh(core_axis_name="c", subcore_axis_name="s")
    return pl.kernel(
        _gather1_body,
        mesh=mesh,
        out_type=jax.ShapeDtypeStruct((B, D2), jnp.float32),
        scratch_types=[
            pltpu.VMEM((NCH, CH), jnp.int32),
            pltpu.VMEM((CH, D2), jnp.float32),
            pltpu.VMEM((CH, D2), jnp.float32),
            pltpu.SemaphoreType.DMA,
            pltpu.SemaphoreType.DMA,
        ],
    )


def _fuse_body(u, i, w1a, w1b, b1, wg, wm, bo, out):
    ug = u[:, :D]
    um = u[:, D:]
    ig = i[:, :D]
    im = i[:, D:]
    h = jnp.dot(um, w1a[...], preferred_element_type=jnp.float32)
    h = h + jnp.dot(im, w1b[...], preferred_element_type=jnp.float32)
    h = jnp.maximum(h + b1[...], 0.0)
    g = ug * ig
    out[...] = (jnp.dot(g, wg[...], preferred_element_type=jnp.float32)
                + jnp.dot(h, wm[...], preferred_element_type=jnp.float32)
                + bo[...])


@functools.lru_cache(maxsize=1)
def _get_fuse():
    return pl.pallas_call(
        _fuse_body,
        grid=(B // BT,),
        in_specs=[
            pl.BlockSpec((BT, D2), lambda i: (i, 0)),
            pl.BlockSpec((BT, D2), lambda i: (i, 0)),
            pl.BlockSpec((D, D), lambda i: (0, 0)),
            pl.BlockSpec((D, D), lambda i: (0, 0)),
            pl.BlockSpec((1, D), lambda i: (0, 0)),
            pl.BlockSpec((D, 1), lambda i: (0, 0)),
            pl.BlockSpec((D, 1), lambda i: (0, 0)),
            pl.BlockSpec((1, 1), lambda i: (0, 0)),
        ],
        out_specs=pl.BlockSpec((BT, 1), lambda i: (i, 0)),
        out_shape=jax.ShapeDtypeStruct((B, 1), jnp.float32),
    )


def kernel(user_indices, item_indices, user_emb_ncf, item_emb_ncf,
           user_emb_mlp, item_emb_mlp, W1, b1, W_out, b_out):
    uidx = user_indices.astype(jnp.int32).reshape(NW, NCH, CH)
    iidx = item_indices.astype(jnp.int32).reshape(NW, NCH, CH)
    nu = user_emb_ncf.shape[0]
    ni = item_emb_ncf.shape[0]
    t_u = _get_packT(nu)(user_emb_ncf.T, user_emb_mlp.T)
    t_i = _get_packT(ni)(item_emb_ncf.T, item_emb_mlp.T)
    g = _get_gather1()
    u = g(uidx, t_u)
    i = g(iidx, t_i)
    return _get_fuse()(u, i, W1[:D], W1[D:], b1.reshape(1, D),
                       W_out[:D], W_out[D:], b_out.reshape(1, 1))
